# Initial kernel scaffold; baseline (speedup 1.0000x reference)
import functools

import jax
import jax.numpy as jnp
from jax import lax
from jax.experimental import pallas as pl
from jax.experimental.pallas import tpu as pltpu
from jax.experimental.pallas import tpu_sc as plsc

N_LV = 8
TBL = 65536
N_PTS = 1048576
NC, NS, LANES = 2, 16, 16
NW = NC * NS
N_CHUNKS = NW // N_LV
PTS_PER_TEC = N_PTS // N_CHUNKS
C = 8192
N_STAGES = PTS_PER_TEC // C
P1 = jnp.uint32(2654435761)
P2 = jnp.uint32(805459861)
MASK = jnp.uint32(TBL - 1)


def _body(xs_hbm, ys_hbm, zs_hbm, tbl_hbm, out_hbm,
          tbl_v, xs_v, ys_v, zs_v, out_v):
    wid = lax.axis_index("s") * NC + lax.axis_index("c")
    level = lax.rem(wid, N_LV)
    chunk = wid // N_LV
    res_f = (16 << level).astype(jnp.float32)

    pltpu.sync_copy(tbl_hbm.at[pl.ds(level * TBL, TBL)], tbl_v)

    def stage(it, _):
        base = chunk * PTS_PER_TEC + it * C
        pltpu.sync_copy(xs_hbm.at[pl.ds(base, C)], xs_v)
        pltpu.sync_copy(ys_hbm.at[pl.ds(base, C)], ys_v)
        pltpu.sync_copy(zs_hbm.at[pl.ds(base, C)], zs_v)

        def vreg(i, _):
            o = i * LANES
            xv = xs_v[pl.ds(o, LANES)]
            yv = ys_v[pl.ds(o, LANES)]
            zv = zs_v[pl.ds(o, LANES)]
            px = xv * res_f
            py = yv * res_f
            pz = zv * res_f
            ix = px.astype(jnp.int32)
            iy = py.astype(jnp.int32)
            iz = pz.astype(jnp.int32)
            fx = px - ix.astype(jnp.float32)
            fy = py - iy.astype(jnp.float32)
            fz = pz - iz.astype(jnp.float32)
            hx0 = ix.astype(jnp.uint32)
            hx1 = hx0 + jnp.uint32(1)
            hy0 = iy.astype(jnp.uint32) * P1
            hy1 = hy0 + P1
            hz0 = iz.astype(jnp.uint32) * P2
            hz1 = hz0 + P2
            hyz = (hy0 ^ hz0, hy0 ^ hz1, hy1 ^ hz0, hy1 ^ hz1)
            one = jnp.float32(1.0)
            wx = (one - fx, fx)
            wy = (one - fy, fy)
            wz = (one - fz, fz)
            wyz = (wy[0] * wz[0], wy[0] * wz[1], wy[1] * wz[0], wy[1] * wz[1])
            acc0 = jnp.zeros((LANES,), jnp.float32)
            acc1 = jnp.zeros((LANES,), jnp.float32)
            for ox in range(2):
                hx = (hx0, hx1)[ox]
                for oyz in range(4):
                    idx = ((hx ^ hyz[oyz]) & MASK).astype(jnp.int32)
                    g = plsc.load_gather(tbl_v, [idx])
                    gb = plsc.bitcast(g, jnp.bfloat16)
                    f0, f1 = plsc.unpack(gb, format=plsc.PackFormat.INTERLEAVED)
                    w = wx[ox] * wyz[oyz]
                    acc0 = acc0 + w * f0
                    acc1 = acc1 + w * f1
            packed = plsc.pack(acc0, acc1, format=plsc.PackFormat.INTERLEAVED)
            out_v[pl.ds(o, LANES)] = plsc.bitcast(packed, jnp.int32)
            return 0

        lax.fori_loop(0, C // LANES, vreg, 0)
        pltpu.sync_copy(out_v, out_hbm.at[pl.ds(level * N_PTS + base, C)])
        return 0

    lax.fori_loop(0, N_STAGES, stage, 0)


_encoder = functools.partial(
    pl.kernel,
    out_type=jax.ShapeDtypeStruct((N_LV * N_PTS,), jnp.int32),
    mesh=plsc.VectorSubcoreMesh(
        core_axis_name="c", subcore_axis_name="s",
        num_cores=NC, num_subcores=NS),
    scratch_types=[
        pltpu.VMEM((TBL,), jnp.int32),
        pltpu.VMEM((C,), jnp.float32),
        pltpu.VMEM((C,), jnp.float32),
        pltpu.VMEM((C,), jnp.float32),
        pltpu.VMEM((C,), jnp.int32),
    ],
)(_body)


def kernel(input, table):
    xt = input.T
    packed_tbl = jax.lax.bitcast_convert_type(
        table.astype(jnp.bfloat16), jnp.int32).reshape(-1)
    words = _encoder(xt[0], xt[1], xt[2], packed_tbl)
    feats = jax.lax.bitcast_convert_type(
        words.reshape(N_LV, N_PTS), jnp.bfloat16)
    return feats.astype(jnp.float32).transpose(1, 0, 2).reshape(N_PTS, 16)


# speedup vs baseline: 162.3112x; 162.3112x over previous
import functools

import jax
import jax.numpy as jnp
from jax import lax
from jax.experimental import pallas as pl
from jax.experimental.pallas import tpu as pltpu
from jax.experimental.pallas import tpu_sc as plsc

N_LV = 8
TBL = 65536
N_PTS = 1048576
NC, NS, LANES = 2, 16, 16
NW = NC * NS
N_CHUNKS = NW // N_LV
PTS_PER_TEC = N_PTS // N_CHUNKS
C = 8192
N_STAGES = PTS_PER_TEC // C
PRIME_Y = 2654435761
PRIME_Z = 805459861


def _body(xs_hbm, ys_hbm, zs_hbm, tbl_hbm, out_hbm,
          tbl_v, xs_v, ys_v, zs_v, out_v):
    p1 = jnp.uint32(PRIME_Y)
    p2 = jnp.uint32(PRIME_Z)
    mask = jnp.uint32(TBL - 1)
    wid = lax.axis_index("s") * NC + lax.axis_index("c")
    level = lax.rem(wid, N_LV)
    chunk = wid // N_LV
    res_f = (16 << level).astype(jnp.float32)

    pltpu.sync_copy(tbl_hbm.at[pl.ds(level * TBL, TBL)], tbl_v)

    def stage(it, _):
        base = chunk * PTS_PER_TEC + it * C
        pltpu.sync_copy(xs_hbm.at[pl.ds(base, C)], xs_v)
        pltpu.sync_copy(ys_hbm.at[pl.ds(base, C)], ys_v)
        pltpu.sync_copy(zs_hbm.at[pl.ds(base, C)], zs_v)

        def vreg(i, _):
            o = i * LANES
            xv = xs_v[pl.ds(o, LANES)]
            yv = ys_v[pl.ds(o, LANES)]
            zv = zs_v[pl.ds(o, LANES)]
            px = xv * res_f
            py = yv * res_f
            pz = zv * res_f
            ix = px.astype(jnp.int32)
            iy = py.astype(jnp.int32)
            iz = pz.astype(jnp.int32)
            fx = px - ix.astype(jnp.float32)
            fy = py - iy.astype(jnp.float32)
            fz = pz - iz.astype(jnp.float32)
            hx0 = ix.astype(jnp.uint32)
            hx1 = hx0 + jnp.uint32(1)
            hy0 = iy.astype(jnp.uint32) * p1
            hy1 = hy0 + p1
            hz0 = iz.astype(jnp.uint32) * p2
            hz1 = hz0 + p2
            hyz = (hy0 ^ hz0, hy0 ^ hz1, hy1 ^ hz0, hy1 ^ hz1)
            one = jnp.float32(1.0)
            wx = (one - fx, fx)
            wy = (one - fy, fy)
            wz = (one - fz, fz)
            wyz = (wy[0] * wz[0], wy[0] * wz[1], wy[1] * wz[0], wy[1] * wz[1])
            acc0 = jnp.zeros((LANES,), jnp.float32)
            acc1 = jnp.zeros((LANES,), jnp.float32)
            for ox in range(2):
                hx = (hx0, hx1)[ox]
                for oyz in range(4):
                    idx = ((hx ^ hyz[oyz]) & mask).astype(jnp.int32)
                    g = plsc.load_gather(tbl_v, [idx])
                    gb = plsc.bitcast(g, jnp.bfloat16)
                    f0, f1 = plsc.unpack(gb, format=plsc.PackFormat.INTERLEAVED)
                    w = wx[ox] * wyz[oyz]
                    acc0 = acc0 + w * f0
                    acc1 = acc1 + w * f1
            packed = plsc.pack(acc0, acc1, format=plsc.PackFormat.INTERLEAVED)
            out_v[pl.ds(o, LANES)] = plsc.bitcast(packed, jnp.int32)
            return 0

        lax.fori_loop(0, C // LANES, vreg, 0)
        pltpu.sync_copy(out_v, out_hbm.at[pl.ds(level * N_PTS + base, C)])
        return 0

    lax.fori_loop(0, N_STAGES, stage, 0)


_encoder = functools.partial(
    pl.kernel,
    out_type=jax.ShapeDtypeStruct((N_LV * N_PTS,), jnp.int32),
    mesh=plsc.VectorSubcoreMesh(
        core_axis_name="c", subcore_axis_name="s",
        num_cores=NC, num_subcores=NS),
    compiler_params=pltpu.CompilerParams(needs_layout_passes=False),
    scratch_types=[
        pltpu.VMEM((TBL,), jnp.int32),
        pltpu.VMEM((C,), jnp.float32),
        pltpu.VMEM((C,), jnp.float32),
        pltpu.VMEM((C,), jnp.float32),
        pltpu.VMEM((C,), jnp.int32),
    ],
)(_body)


def kernel(input, table):
    xt = input.T
    packed_tbl = jax.lax.bitcast_convert_type(
        table.astype(jnp.bfloat16), jnp.int32).reshape(-1)
    words = _encoder(xt[0], xt[1], xt[2], packed_tbl)
    feats = jax.lax.bitcast_convert_type(
        words.reshape(N_LV, N_PTS), jnp.bfloat16)
    return feats.astype(jnp.float32).transpose(1, 0, 2).reshape(N_PTS, 16)


# speedup vs baseline: 162.3582x; 1.0003x over previous
import functools

import jax
import jax.numpy as jnp
from jax import lax
from jax.experimental import pallas as pl
from jax.experimental.pallas import tpu as pltpu
from jax.experimental.pallas import tpu_sc as plsc

N_LV = 8
TBL = 65536
N_PTS = 1048576
NC, NS, LANES = 2, 16, 16
NW = NC * NS
N_CHUNKS = NW // N_LV
PTS_PER_TEC = N_PTS // N_CHUNKS
C = 8192
N_STAGES = PTS_PER_TEC // C
PRIME_Y = 2654435761
PRIME_Z = 805459861


def _body(xs_hbm, ys_hbm, zs_hbm, tbl_hbm, out_hbm,
          tbl_v, xs_v, ys_v, zs_v, out_v):
    p1 = jnp.uint32(PRIME_Y)
    p2 = jnp.uint32(PRIME_Z)
    mask = jnp.uint32(TBL - 1)
    wid = lax.axis_index("s") * NC + lax.axis_index("c")
    level = lax.rem(wid, N_LV)
    chunk = wid // N_LV
    res_f = (16 << level).astype(jnp.float32)

    pltpu.sync_copy(tbl_hbm.at[pl.ds(level * TBL, TBL)], tbl_v)

    def stage(it, _):
        base = chunk * PTS_PER_TEC + it * C
        pltpu.sync_copy(xs_hbm.at[pl.ds(base, C)], xs_v)
        pltpu.sync_copy(ys_hbm.at[pl.ds(base, C)], ys_v)
        pltpu.sync_copy(zs_hbm.at[pl.ds(base, C)], zs_v)

        def vreg(i, _):
            o = i * LANES
            xv = xs_v[pl.ds(o, LANES)]
            yv = ys_v[pl.ds(o, LANES)]
            zv = zs_v[pl.ds(o, LANES)]
            px = xv * res_f
            py = yv * res_f
            pz = zv * res_f
            ix = px.astype(jnp.int32)
            iy = py.astype(jnp.int32)
            iz = pz.astype(jnp.int32)
            fx = px - ix.astype(jnp.float32)
            fy = py - iy.astype(jnp.float32)
            fz = pz - iz.astype(jnp.float32)
            hx0 = ix.astype(jnp.uint32)
            hx1 = hx0 + jnp.uint32(1)
            hy0 = iy.astype(jnp.uint32) * p1
            hy1 = hy0 + p1
            hz0 = iz.astype(jnp.uint32) * p2
            hz1 = hz0 + p2
            hyz = (hy0 ^ hz0, hy0 ^ hz1, hy1 ^ hz0, hy1 ^ hz1)
            one = jnp.float32(1.0)
            wx = (one - fx, fx)
            wy = (one - fy, fy)
            wz = (one - fz, fz)
            wyz = (wy[0] * wz[0], wy[0] * wz[1], wy[1] * wz[0], wy[1] * wz[1])
            acc0 = jnp.zeros((LANES,), jnp.float32)
            acc1 = jnp.zeros((LANES,), jnp.float32)
            for ox in range(2):
                hx = (hx0, hx1)[ox]
                for oyz in range(4):
                    idx = ((hx ^ hyz[oyz]) & mask).astype(jnp.int32)
                    g = plsc.load_gather(tbl_v, [idx])
                    gb = plsc.bitcast(g, jnp.bfloat16)
                    f0, f1 = plsc.unpack(gb, format=plsc.PackFormat.INTERLEAVED)
                    w = wx[ox] * wyz[oyz]
                    acc0 = acc0 + w * f0
                    acc1 = acc1 + w * f1
            packed = plsc.pack(acc0, acc1, format=plsc.PackFormat.INTERLEAVED)
            out_v[pl.ds(o, LANES)] = plsc.bitcast(packed, jnp.int32)
            return 0

        lax.fori_loop(0, C // LANES, vreg, 0)
        pltpu.sync_copy(out_v, out_hbm.at[pl.ds(level * N_PTS + base, C)])
        return 0

    lax.fori_loop(0, N_STAGES, stage, 0)


_encoder = functools.partial(
    pl.kernel,
    out_type=jax.ShapeDtypeStruct((N_LV * N_PTS,), jnp.int32),
    mesh=plsc.VectorSubcoreMesh(
        core_axis_name="c", subcore_axis_name="s",
        num_cores=NC, num_subcores=NS),
    compiler_params=pltpu.CompilerParams(needs_layout_passes=False),
    scratch_types=[
        pltpu.VMEM((TBL,), jnp.int32),
        pltpu.VMEM((C,), jnp.float32),
        pltpu.VMEM((C,), jnp.float32),
        pltpu.VMEM((C,), jnp.float32),
        pltpu.VMEM((C,), jnp.int32),
    ],
)(_body)


def kernel(input, table):
    xt = input.T
    packed_tbl = jax.lax.bitcast_convert_type(
        table.astype(jnp.bfloat16), jnp.int32).reshape(-1)
    words = _encoder(xt[0], xt[1], xt[2], packed_tbl)
    wt = words.reshape(N_LV, N_PTS).T
    feats = jax.lax.bitcast_convert_type(wt, jnp.bfloat16)
    return feats.astype(jnp.float32).reshape(N_PTS, 16)
